# baseline (speedup 1.0000x reference)
import functools

import jax
import jax.numpy as jnp
from jax import lax
from jax.experimental import pallas as pl
from jax.experimental.pallas import tpu as pltpu
from jax.experimental.pallas import tpu_sc as plsc

N_FACTORS = 64
WPR = N_FACTORS // 2
QUAD = 4 * WPR
BATCH = 16384

_info = plsc.get_sparse_core_info()
NC = _info.num_cores
NS = _info.num_subcores
LANES = _info.num_lanes
NW = NC * NS
BPW = BATCH // NW
GCHUNK = 128
CH = 256
NPASS = BPW // CH

_HI_MASK = -65536


def _decode_mul(wu, wb):
  ulo = plsc.bitcast(lax.shift_left(wu, 16), jnp.float32)
  blo = plsc.bitcast(lax.shift_left(wb, 16), jnp.float32)
  uhi = plsc.bitcast(lax.bitwise_and(wu, _HI_MASK), jnp.float32)
  bhi = plsc.bitcast(lax.bitwise_and(wb, _HI_MASK), jnp.float32)
  return ulo * blo + uhi * bhi


def _body(user_hbm, book_hbm, uemb_hbm, bemb_hbm, out_hbm,
          uidx_v, bidx_v, uquad_v, bquad_v, urows_v, brows_v, out_v,
          sem_u, sem_b):
  wid = lax.axis_index("s") * NC + lax.axis_index("c")
  base = wid * BPW

  pltpu.sync_copy(user_hbm.at[pl.ds(base, BPW)], uidx_v)
  pltpu.sync_copy(book_hbm.at[pl.ds(base, BPW)], bidx_v)

  for i in range(BPW // LANES):
    sl = pl.ds(i * LANES, LANES)
    uquad_v[sl] = lax.shift_right_logical(uidx_v[sl], 2)
    bquad_v[sl] = lax.shift_right_logical(bidx_v[sl], 2)

  lane = lax.iota(jnp.int32, LANES)

  for p in range(NPASS):
    copies = []
    for k in range(CH // GCHUNK):
      isl = pl.ds(p * CH + k * GCHUNK, GCHUNK)
      dsl = pl.ds(k * GCHUNK, GCHUNK)
      copies.append(pltpu.async_copy(
          uemb_hbm.at[uquad_v.at[isl]], urows_v.at[dsl], sem_u))
      copies.append(pltpu.async_copy(
          bemb_hbm.at[bquad_v.at[isl]], brows_v.at[dsl], sem_b))
    for c in copies:
      c.wait()

    def group(g, carry):
      gsl = pl.ds(p * CH + g * LANES, LANES)
      hu_vec = (uidx_v[gsl] & 3) * WPR
      hb_vec = (bidx_v[gsl] & 3) * WPR
      res = jnp.zeros((LANES,), jnp.float32)
      for c in range(LANES):
        onehot = lane == c
        hu = jnp.sum(jnp.where(onehot, hu_vec, 0))
        hb = jnp.sum(jnp.where(onehot, hb_vec, 0))
        r = g * LANES + c
        acc = jnp.zeros((LANES,), jnp.float32)
        for k in range(WPR // LANES):
          wu = urows_v[r, pl.ds(hu + k * LANES, LANES)]
          wb = brows_v[r, pl.ds(hb + k * LANES, LANES)]
          acc = acc + _decode_mul(wu, wb)
        res = jnp.where(onehot, jnp.sum(acc), res)
      out_v[pl.ds(g * LANES, LANES)] = res
      return carry

    lax.fori_loop(0, CH // LANES, group, 0)
    pltpu.sync_copy(out_v, out_hbm.at[pl.ds(base + p * CH, CH)])


def _pack_table(emb):
  n = emb.shape[0]
  b = emb.astype(jnp.bfloat16).reshape(n, WPR, 2)
  w = lax.bitcast_convert_type(b, jnp.int32)
  return w.reshape(n // 4, QUAD)


@jax.jit
def kernel(user, book, user_emb, book_emb):
  upk = _pack_table(user_emb)
  bpk = _pack_table(book_emb)
  mesh = plsc.VectorSubcoreMesh(core_axis_name="c", subcore_axis_name="s")
  run = functools.partial(
      pl.kernel,
      out_type=jax.ShapeDtypeStruct((BATCH,), jnp.float32),
      mesh=mesh,
      compiler_params=pltpu.CompilerParams(needs_layout_passes=False),
      scratch_types=[
          pltpu.VMEM((BPW,), jnp.int32),
          pltpu.VMEM((BPW,), jnp.int32),
          pltpu.VMEM((BPW,), jnp.int32),
          pltpu.VMEM((BPW,), jnp.int32),
          pltpu.VMEM((CH, QUAD), jnp.int32),
          pltpu.VMEM((CH, QUAD), jnp.int32),
          pltpu.VMEM((CH,), jnp.float32),
          pltpu.SemaphoreType.DMA,
          pltpu.SemaphoreType.DMA,
      ],
  )(_body)
  return run(user.astype(jnp.int32), book.astype(jnp.int32), upk, bpk)


# speedup vs baseline: 3.0737x; 3.0737x over previous
import functools

import jax
import jax.numpy as jnp
from jax import lax
from jax.experimental import pallas as pl
from jax.experimental.pallas import tpu as pltpu
from jax.experimental.pallas import tpu_sc as plsc

N_FACTORS = 64
PADW = 128
BATCH = 16384

_info = plsc.get_sparse_core_info()
NC = _info.num_cores
NS = _info.num_subcores
LANES = _info.num_lanes
NW = NC * NS
BPW = BATCH // NW
GCHUNK = 128
CH = 256
NPASS = BPW // CH


def _body(user_hbm, book_hbm, uemb_hbm, bemb_hbm, out_hbm,
          uidx_v, bidx_v, urows_v, brows_v, out_v, sem_u, sem_b):
  wid = lax.axis_index("s") * NC + lax.axis_index("c")
  base = wid * BPW

  pltpu.sync_copy(user_hbm.at[pl.ds(base, BPW)], uidx_v)
  pltpu.sync_copy(book_hbm.at[pl.ds(base, BPW)], bidx_v)

  lane = lax.iota(jnp.int32, LANES)
  one = jnp.ones((LANES,), jnp.int32)

  for p in range(NPASS):
    copies = []
    for k in range(CH // GCHUNK):
      isl = pl.ds(p * CH + k * GCHUNK, GCHUNK)
      dsl = pl.ds(k * GCHUNK, GCHUNK)
      copies.append(pltpu.async_copy(
          uemb_hbm.at[uidx_v.at[isl]], urows_v.at[dsl], sem_u))
      copies.append(pltpu.async_copy(
          bemb_hbm.at[bidx_v.at[isl]], brows_v.at[dsl], sem_b))
    for c in copies:
      c.wait()

    def group(g, carry):
      rows = g * LANES + lane
      col = jnp.zeros((LANES,), jnp.int32)
      acc = jnp.zeros((LANES,), jnp.float32)
      for j in range(N_FACTORS):
        u = plsc.load_gather(urows_v, [rows, col])
        b = plsc.load_gather(brows_v, [rows, col])
        acc = acc + u * b
        if j + 1 < N_FACTORS:
          col = col + one
      out_v[pl.ds(g * LANES, LANES)] = acc
      return carry

    lax.fori_loop(0, CH // LANES, group, 0)
    pltpu.sync_copy(out_v, out_hbm.at[pl.ds(base + p * CH, CH)])


@jax.jit
def kernel(user, book, user_emb, book_emb):
  pad = ((0, 0), (0, PADW - N_FACTORS))
  up = jnp.pad(user_emb, pad)
  bp = jnp.pad(book_emb, pad)
  mesh = plsc.VectorSubcoreMesh(core_axis_name="c", subcore_axis_name="s")
  run = functools.partial(
      pl.kernel,
      out_type=jax.ShapeDtypeStruct((BATCH,), jnp.float32),
      mesh=mesh,
      compiler_params=pltpu.CompilerParams(needs_layout_passes=False),
      scratch_types=[
          pltpu.VMEM((BPW,), jnp.int32),
          pltpu.VMEM((BPW,), jnp.int32),
          pltpu.VMEM((CH, PADW), jnp.float32),
          pltpu.VMEM((CH, PADW), jnp.float32),
          pltpu.VMEM((CH,), jnp.float32),
          pltpu.SemaphoreType.DMA,
          pltpu.SemaphoreType.DMA,
      ],
  )(_body)
  return run(user.astype(jnp.int32), book.astype(jnp.int32), up, bp)


# speedup vs baseline: 3.0751x; 1.0005x over previous
import functools

import jax
import jax.numpy as jnp
from jax import lax
from jax.experimental import pallas as pl
from jax.experimental.pallas import tpu as pltpu
from jax.experimental.pallas import tpu_sc as plsc

N_FACTORS = 64
PADW = 128
BATCH = 16384

_info = plsc.get_sparse_core_info()
NC = _info.num_cores
NS = _info.num_subcores
LANES = _info.num_lanes
NW = NC * NS
BPW = BATCH // NW
GCHUNK = 128
CH = 256
NPASS = BPW // CH


def _body(user_hbm, book_hbm, uemb_hbm, bemb_hbm, out_hbm,
          uidx_v, bidx_v, urows_v, brows_v, out_v, sem_u, sem_b):
  wid = lax.axis_index("s") * NC + lax.axis_index("c")
  base = wid * BPW

  pltpu.sync_copy(user_hbm.at[pl.ds(base, BPW)], uidx_v)
  pltpu.sync_copy(book_hbm.at[pl.ds(base, BPW)], bidx_v)

  lane = lax.iota(jnp.int32, LANES)
  one = jnp.ones((LANES,), jnp.int32)

  for p in range(NPASS):
    copies = []
    for k in range(CH // GCHUNK):
      isl = pl.ds(p * CH + k * GCHUNK, GCHUNK)
      dsl = pl.ds(k * GCHUNK, GCHUNK)
      copies.append(pltpu.async_copy(
          uemb_hbm.at[uidx_v.at[isl]], urows_v.at[dsl], sem_u))
      copies.append(pltpu.async_copy(
          bemb_hbm.at[bidx_v.at[isl]], brows_v.at[dsl], sem_b))
    for c in copies:
      c.wait()

    def group(g, carry):
      rows = g * LANES + lane
      col = jnp.zeros((LANES,), jnp.int32)
      acc = jnp.zeros((LANES,), jnp.float32)
      for j in range(N_FACTORS):
        u = plsc.load_gather(urows_v, [rows, col])
        b = plsc.load_gather(brows_v, [rows, col])
        acc = acc + u * b
        if j + 1 < N_FACTORS:
          col = col + one
      out_v[pl.ds(g * LANES, LANES)] = acc
      return carry

    lax.fori_loop(0, CH // LANES, group, 0)
    pltpu.sync_copy(out_v, out_hbm.at[pl.ds(base + p * CH, CH)])


@jax.jit
def kernel(user, book, user_emb, book_emb):
  pad = ((0, 0), (0, PADW - N_FACTORS))
  z = (user[0] & 0).astype(jnp.float32)
  up = jnp.pad(user_emb, pad) + z
  bp = jnp.pad(book_emb, pad)
  mesh = plsc.VectorSubcoreMesh(core_axis_name="c", subcore_axis_name="s")
  run = functools.partial(
      pl.kernel,
      out_type=jax.ShapeDtypeStruct((BATCH,), jnp.float32),
      mesh=mesh,
      compiler_params=pltpu.CompilerParams(needs_layout_passes=False),
      scratch_types=[
          pltpu.VMEM((BPW,), jnp.int32),
          pltpu.VMEM((BPW,), jnp.int32),
          pltpu.VMEM((CH, PADW), jnp.float32),
          pltpu.VMEM((CH, PADW), jnp.float32),
          pltpu.VMEM((CH,), jnp.float32),
          pltpu.SemaphoreType.DMA,
          pltpu.SemaphoreType.DMA,
      ],
  )(_body)
  return run(user.astype(jnp.int32), book.astype(jnp.int32), up, bp)
